# baseline (speedup 1.0000x reference)
import functools

import jax
import jax.numpy as jnp
from jax import lax
from jax.experimental import pallas as pl
from jax.experimental.pallas import tpu as pltpu
from jax.experimental.pallas import tpu_sc as plsc

N = 10000
D = 128
E = 320000
NS = 16
CH = 128
NBUF = 4
ABUF = 2
CPS = 160
CPA = NS * CPS
TOT = 2 * CPA
EPAD = CPA * CH
NDUMP = 64
NP = N + NDUMP
RPT = (N // NS) // 8 * 8
RTL = N - NS * RPT
BLK = 1000


def _deg_body(comb_ref, zeros_ref, ones_ref, out_ref,
              iv0, iv1, iv2, iv3, ones_v, s0, s1, s2, s3, acc_s):
    ivs = (iv0, iv1, iv2, iv3)
    sems = (s0, s1, s2, s3)
    c = lax.axis_index("c")
    s = lax.axis_index("s")
    pltpu.sync_copy(ones_ref, ones_v)
    pltpu.sync_copy(zeros_ref.at[pl.ds(s * RPT, RPT)],
                    acc_s.at[pl.ds(s * RPT, RPT)])

    @pl.when(s == NS - 1)
    def _():
        pltpu.sync_copy(zeros_ref.at[pl.ds(NS * RPT, RTL)],
                        acc_s.at[pl.ds(NS * RPT, RTL)])

    plsc.subcore_barrier()
    cbase = c * CPA + s * CPS
    for b in range(NBUF):
        pltpu.async_copy(comb_ref.at[cbase + b], ivs[b], sems[b])

    @pl.loop(0, CPS // NBUF - 1)
    def _(t):
        for b in range(NBUF):
            pltpu.make_async_copy(comb_ref.at[0], ivs[b], sems[b]).wait()
            pltpu.sync_copy(ones_v, acc_s.at[ivs[b].at[1]], add=True)
            pltpu.async_copy(comb_ref.at[cbase + (t + 1) * NBUF + b],
                             ivs[b], sems[b])

    for b in range(NBUF):
        pltpu.make_async_copy(comb_ref.at[0], ivs[b], sems[b]).wait()
        pltpu.sync_copy(ones_v, acc_s.at[ivs[b].at[1]], add=True)
    plsc.subcore_barrier()
    pltpu.sync_copy(acc_s.at[pl.ds(s * RPT, RPT)],
                    out_ref.at[c, pl.ds(s * RPT, RPT)])

    @pl.when(s == NS - 1)
    def _():
        pltpu.sync_copy(acc_s.at[pl.ds(NS * RPT, RTL)],
                        out_ref.at[c, pl.ds(NS * RPT, RTL)])


def _agg_body(y_ref, comb_ref, out_ref,
              iv0, iv1, r0, r1, g0, g1, t0, t1, acc_s):
    ivs = (iv0, iv1)
    rows = (r0, r1)
    gsems = (g0, g1)
    ssems = (t0, t1)
    c = lax.axis_index("c")
    s = lax.axis_index("s")
    pltpu.sync_copy(y_ref.at[pl.ds(c * N + s * RPT, RPT)],
                    acc_s.at[pl.ds(s * RPT, RPT)])

    @pl.when(s == NS - 1)
    def _():
        pltpu.sync_copy(y_ref.at[pl.ds(c * N + NS * RPT, RTL)],
                        acc_s.at[pl.ds(NS * RPT, RTL)])

    plsc.subcore_barrier()
    cbase = c * CPA + s * CPS
    for b in range(ABUF):
        pltpu.sync_copy(comb_ref.at[cbase + b], ivs[b])
        pltpu.async_copy(y_ref.at[ivs[b].at[0]], rows[b], gsems[b])

    @pl.loop(0, CPS // ABUF - 1)
    def _(t):
        for b in range(ABUF):
            pltpu.make_async_copy(y_ref.at[pl.ds(0, CH)], rows[b],
                                  gsems[b]).wait()
            pltpu.async_copy(rows[b], acc_s.at[ivs[b].at[1]], ssems[b],
                             add=True)
        for b in range(ABUF):
            pltpu.make_async_copy(y_ref.at[pl.ds(0, CH)],
                                  acc_s.at[pl.ds(0, CH)], ssems[b]).wait()
            pltpu.sync_copy(comb_ref.at[cbase + (t + 1) * ABUF + b], ivs[b])
            pltpu.async_copy(y_ref.at[ivs[b].at[0]], rows[b], gsems[b])

    for b in range(ABUF):
        pltpu.make_async_copy(y_ref.at[pl.ds(0, CH)], rows[b],
                              gsems[b]).wait()
        pltpu.sync_copy(rows[b], acc_s.at[ivs[b].at[1]], add=True)
    plsc.subcore_barrier()
    pltpu.sync_copy(acc_s.at[pl.ds(s * RPT, RPT)],
                    out_ref.at[c, pl.ds(s * RPT, RPT)])

    @pl.when(s == NS - 1)
    def _():
        pltpu.sync_copy(acc_s.at[pl.ds(NS * RPT, RTL)],
                        out_ref.at[c, pl.ds(NS * RPT, RTL)])


def _scale_body(x_ref, w_ref, deg_ref, y_ref):
    xw = jnp.dot(x_ref[...], w_ref[...], preferred_element_type=jnp.float32)
    d0 = lax.rsqrt(deg_ref[0, :, 0:1] + 1.0)
    d1 = lax.rsqrt(deg_ref[1, :, 0:1] + 1.0)
    y_ref[0] = d0 * xw
    y_ref[1] = d1 * xw


_scale_call = pl.pallas_call(
    _scale_body,
    grid=(N // BLK,),
    in_specs=[
        pl.BlockSpec((BLK, D), lambda i: (i, 0)),
        pl.BlockSpec((D, D), lambda i: (0, 0)),
        pl.BlockSpec((2, BLK, D), lambda i: (0, i, 0)),
    ],
    out_specs=pl.BlockSpec((2, BLK, D), lambda i: (0, i, 0)),
    out_shape=jax.ShapeDtypeStruct((2, N, D), jnp.float32),
)


def _final_body(agg_ref, deg_ref, b_ref, out_ref):
    bv = b_ref[0]
    d0 = lax.rsqrt(deg_ref[0, :, 0:1] + 1.0)
    d1 = lax.rsqrt(deg_ref[1, :, 0:1] + 1.0)
    out_ref[:, :D] = d0 * agg_ref[0] + bv
    out_ref[:, D:] = d1 * agg_ref[1] + bv


_final_call = pl.pallas_call(
    _final_body,
    grid=(N // BLK,),
    in_specs=[
        pl.BlockSpec((2, BLK, D), lambda i: (0, i, 0)),
        pl.BlockSpec((2, BLK, D), lambda i: (0, i, 0)),
        pl.BlockSpec((1, D), lambda i: (0, 0)),
    ],
    out_specs=pl.BlockSpec((BLK, 2 * D), lambda i: (i, 0)),
    out_shape=jax.ShapeDtypeStruct((N, 2 * D), jnp.float32),
)


@functools.cache
def _sc_kernels():
    mesh = plsc.VectorSubcoreMesh(core_axis_name="c", subcore_axis_name="s")
    deg_kernel = pl.kernel(
        _deg_body,
        mesh=mesh,
        out_type=jax.ShapeDtypeStruct((2, N, D), jnp.float32),
        scratch_types=[
            pltpu.VMEM((2, CH), jnp.int32),
            pltpu.VMEM((2, CH), jnp.int32),
            pltpu.VMEM((2, CH), jnp.int32),
            pltpu.VMEM((2, CH), jnp.int32),
            pltpu.VMEM((CH, D), jnp.float32),
            pltpu.SemaphoreType.DMA,
            pltpu.SemaphoreType.DMA,
            pltpu.SemaphoreType.DMA,
            pltpu.SemaphoreType.DMA,
            pltpu.VMEM_SHARED((NP, D), jnp.float32),
        ],
    )
    agg_kernel = pl.kernel(
        _agg_body,
        mesh=mesh,
        out_type=jax.ShapeDtypeStruct((2, N, D), jnp.float32),
        scratch_types=[
            pltpu.VMEM((2, CH), jnp.int32),
            pltpu.VMEM((2, CH), jnp.int32),
            pltpu.VMEM((CH, D), jnp.float32),
            pltpu.VMEM((CH, D), jnp.float32),
            pltpu.SemaphoreType.DMA,
            pltpu.SemaphoreType.DMA,
            pltpu.SemaphoreType.DMA,
            pltpu.SemaphoreType.DMA,
            pltpu.VMEM_SHARED((NP, D), jnp.float32),
        ],
    )
    return deg_kernel, agg_kernel


def kernel(x, edge_index_list, W, b):
    deg_kernel, agg_kernel = _sc_kernels()
    ei = edge_index_list.astype(jnp.int32)
    src = ei[:, 0, :]
    dst = ei[:, 1, :]
    srcoff = src + jnp.arange(2, dtype=jnp.int32)[:, None] * N
    pad_src = jnp.broadcast_to(jnp.array([[0], [N]], jnp.int32), (2, EPAD - E))
    srcoff_p = jnp.concatenate([srcoff, pad_src], axis=1)
    pad_dst = N + jnp.arange(EPAD - E, dtype=jnp.int32) % NDUMP
    dst_p = jnp.concatenate(
        [dst, jnp.broadcast_to(pad_dst, (2, EPAD - E))], axis=1)
    comb = jnp.stack([srcoff_p.reshape(2, CPA, CH),
                      dst_p.reshape(2, CPA, CH)], axis=2).reshape(TOT, 2, CH)
    zerosd = jnp.zeros((N, D), jnp.float32)
    onesd = jnp.ones((CH, D), jnp.float32)

    degp = deg_kernel(comb, zerosd, onesd)
    y = _scale_call(x, W, degp)
    agg = agg_kernel(y.reshape(2 * N, D), comb)
    return _final_call(agg, degp, b.reshape(1, D))


# speedup vs baseline: 1.0963x; 1.0963x over previous
import functools

import jax
import jax.numpy as jnp
from jax import lax
from jax.experimental import pallas as pl
from jax.experimental.pallas import tpu as pltpu
from jax.experimental.pallas import tpu_sc as plsc

N = 10000
D = 128
E = 320000
NS = 16
CH = 128
NBUF = 4
ABUF = 2
CPS = 160
CPA = NS * CPS
TOT = 2 * CPA
EPAD = CPA * CH
NDUMP = 64
NP = N + NDUMP
RPT = (N // NS) // 8 * 8
RTL = N - NS * RPT
BLK = 1000


def _deg_body(comb_ref, zeros_ref, ones_ref, out_ref,
              iv0, iv1, iv2, iv3, ones_v, s0, s1, s2, s3, acc_s):
    ivs = (iv0, iv1, iv2, iv3)
    sems = (s0, s1, s2, s3)
    c = lax.axis_index("c")
    s = lax.axis_index("s")
    pltpu.sync_copy(ones_ref, ones_v)
    pltpu.sync_copy(zeros_ref.at[pl.ds(s * RPT, RPT)],
                    acc_s.at[pl.ds(s * RPT, RPT)])

    @pl.when(s == NS - 1)
    def _():
        pltpu.sync_copy(zeros_ref.at[pl.ds(NS * RPT, RTL)],
                        acc_s.at[pl.ds(NS * RPT, RTL)])

    plsc.subcore_barrier()
    cbase = c * CPA + s * CPS
    for b in range(NBUF):
        pltpu.async_copy(comb_ref.at[cbase + b], ivs[b], sems[b])

    @pl.loop(0, CPS // NBUF - 1)
    def _(t):
        for b in range(NBUF):
            pltpu.make_async_copy(comb_ref.at[0], ivs[b], sems[b]).wait()
            pltpu.sync_copy(ones_v, acc_s.at[ivs[b].at[1]], add=True)
            pltpu.async_copy(comb_ref.at[cbase + (t + 1) * NBUF + b],
                             ivs[b], sems[b])

    for b in range(NBUF):
        pltpu.make_async_copy(comb_ref.at[0], ivs[b], sems[b]).wait()
        pltpu.sync_copy(ones_v, acc_s.at[ivs[b].at[1]], add=True)
    plsc.subcore_barrier()
    pltpu.sync_copy(acc_s.at[pl.ds(s * RPT, RPT)],
                    out_ref.at[c, pl.ds(s * RPT, RPT)])

    @pl.when(s == NS - 1)
    def _():
        pltpu.sync_copy(acc_s.at[pl.ds(NS * RPT, RTL)],
                        out_ref.at[c, pl.ds(NS * RPT, RTL)])


def _agg_body(y_ref, comb_ref, out_ref,
              iv0, iv1, iv2, iv3, r0, r1,
              i0, i1, i2, i3, g0, g1, acc_s):
    ivs = (iv0, iv1, iv2, iv3)
    rows = (r0, r1)
    isems = (i0, i1, i2, i3)
    gsems = (g0, g1)
    c = lax.axis_index("c")
    s = lax.axis_index("s")
    pltpu.sync_copy(y_ref.at[pl.ds(c * N + s * RPT, RPT)],
                    acc_s.at[pl.ds(s * RPT, RPT)])

    @pl.when(s == NS - 1)
    def _():
        pltpu.sync_copy(y_ref.at[pl.ds(c * N + NS * RPT, RTL)],
                        acc_s.at[pl.ds(NS * RPT, RTL)])

    plsc.subcore_barrier()
    cbase = c * CPA + s * CPS

    def wait_idx(b):
        pltpu.make_async_copy(comb_ref.at[0], ivs[b], isems[b]).wait()

    def wait_gather(rb):
        pltpu.make_async_copy(y_ref.at[pl.ds(0, CH)], rows[rb],
                              gsems[rb]).wait()

    for b in range(4):
        pltpu.async_copy(comb_ref.at[cbase + b], ivs[b], isems[b])
    for b in range(2):
        wait_idx(b)
        pltpu.async_copy(y_ref.at[ivs[b].at[0]], rows[b], gsems[b])

    @pl.loop(0, CPS // 4)
    def _(t):
        for b in range(4):
            rb = b % 2
            wait_gather(rb)
            pltpu.sync_copy(rows[rb], acc_s.at[ivs[b].at[1]], add=True)
            pltpu.async_copy(comb_ref.at[cbase + 4 * t + b + 4],
                             ivs[b], isems[b])
            wait_idx((b + 2) % 4)
            pltpu.async_copy(y_ref.at[ivs[(b + 2) % 4].at[0]],
                             rows[rb], gsems[rb])

    for b in range(2):
        wait_gather(b)
    for b in (2, 3):
        wait_idx(b)
    plsc.subcore_barrier()
    pltpu.sync_copy(acc_s.at[pl.ds(s * RPT, RPT)],
                    out_ref.at[c, pl.ds(s * RPT, RPT)])

    @pl.when(s == NS - 1)
    def _():
        pltpu.sync_copy(acc_s.at[pl.ds(NS * RPT, RTL)],
                        out_ref.at[c, pl.ds(NS * RPT, RTL)])


def _scale_body(x_ref, w_ref, deg_ref, y_ref):
    xw = jnp.dot(x_ref[...], w_ref[...], preferred_element_type=jnp.float32)
    d0 = lax.rsqrt(deg_ref[0, :, 0:1] + 1.0)
    d1 = lax.rsqrt(deg_ref[1, :, 0:1] + 1.0)
    y_ref[0] = d0 * xw
    y_ref[1] = d1 * xw


_scale_call = pl.pallas_call(
    _scale_body,
    grid=(N // BLK,),
    in_specs=[
        pl.BlockSpec((BLK, D), lambda i: (i, 0)),
        pl.BlockSpec((D, D), lambda i: (0, 0)),
        pl.BlockSpec((2, BLK, D), lambda i: (0, i, 0)),
    ],
    out_specs=pl.BlockSpec((2, BLK, D), lambda i: (0, i, 0)),
    out_shape=jax.ShapeDtypeStruct((2, N, D), jnp.float32),
)


def _final_body(agg_ref, deg_ref, b_ref, out_ref):
    bv = b_ref[0]
    d0 = lax.rsqrt(deg_ref[0, :, 0:1] + 1.0)
    d1 = lax.rsqrt(deg_ref[1, :, 0:1] + 1.0)
    out_ref[:, :D] = d0 * agg_ref[0] + bv
    out_ref[:, D:] = d1 * agg_ref[1] + bv


_final_call = pl.pallas_call(
    _final_body,
    grid=(N // BLK,),
    in_specs=[
        pl.BlockSpec((2, BLK, D), lambda i: (0, i, 0)),
        pl.BlockSpec((2, BLK, D), lambda i: (0, i, 0)),
        pl.BlockSpec((1, D), lambda i: (0, 0)),
    ],
    out_specs=pl.BlockSpec((BLK, 2 * D), lambda i: (i, 0)),
    out_shape=jax.ShapeDtypeStruct((N, 2 * D), jnp.float32),
)


@functools.cache
def _sc_kernels():
    mesh = plsc.VectorSubcoreMesh(core_axis_name="c", subcore_axis_name="s")
    deg_kernel = pl.kernel(
        _deg_body,
        mesh=mesh,
        out_type=jax.ShapeDtypeStruct((2, N, D), jnp.float32),
        scratch_types=[
            pltpu.VMEM((2, CH), jnp.int32),
            pltpu.VMEM((2, CH), jnp.int32),
            pltpu.VMEM((2, CH), jnp.int32),
            pltpu.VMEM((2, CH), jnp.int32),
            pltpu.VMEM((CH, D), jnp.float32),
            pltpu.SemaphoreType.DMA,
            pltpu.SemaphoreType.DMA,
            pltpu.SemaphoreType.DMA,
            pltpu.SemaphoreType.DMA,
            pltpu.VMEM_SHARED((NP, D), jnp.float32),
        ],
    )
    agg_kernel = pl.kernel(
        _agg_body,
        mesh=mesh,
        out_type=jax.ShapeDtypeStruct((2, N, D), jnp.float32),
        scratch_types=[
            pltpu.VMEM((2, CH), jnp.int32),
            pltpu.VMEM((2, CH), jnp.int32),
            pltpu.VMEM((2, CH), jnp.int32),
            pltpu.VMEM((2, CH), jnp.int32),
            pltpu.VMEM((CH, D), jnp.float32),
            pltpu.VMEM((CH, D), jnp.float32),
            pltpu.SemaphoreType.DMA,
            pltpu.SemaphoreType.DMA,
            pltpu.SemaphoreType.DMA,
            pltpu.SemaphoreType.DMA,
            pltpu.SemaphoreType.DMA,
            pltpu.SemaphoreType.DMA,
            pltpu.VMEM_SHARED((NP, D), jnp.float32),
        ],
    )
    return deg_kernel, agg_kernel


def kernel(x, edge_index_list, W, b):
    deg_kernel, agg_kernel = _sc_kernels()
    ei = edge_index_list.astype(jnp.int32)
    src = ei[:, 0, :]
    dst = ei[:, 1, :]
    srcoff = src + jnp.arange(2, dtype=jnp.int32)[:, None] * N
    pad_src = jnp.broadcast_to(jnp.array([[0], [N]], jnp.int32), (2, EPAD - E))
    srcoff_p = jnp.concatenate([srcoff, pad_src], axis=1)
    pad_dst = N + jnp.arange(EPAD - E, dtype=jnp.int32) % NDUMP
    dst_p = jnp.concatenate(
        [dst, jnp.broadcast_to(pad_dst, (2, EPAD - E))], axis=1)
    comb = jnp.stack([srcoff_p.reshape(2, CPA, CH),
                      dst_p.reshape(2, CPA, CH)], axis=2).reshape(TOT, 2, CH)
    comb = jnp.pad(comb, ((0, 8), (0, 0), (0, 0)))
    zerosd = jnp.zeros((N, D), jnp.float32)
    onesd = jnp.ones((CH, D), jnp.float32)

    degp = deg_kernel(comb, zerosd, onesd)
    y = _scale_call(x, W, degp)
    agg = agg_kernel(y.reshape(2 * N, D), comb)
    return _final_call(agg, degp, b.reshape(1, D))
